# baseline (speedup 1.0000x reference)
import functools
import math

import jax
import jax.numpy as jnp
from jax.experimental import pallas as pl
from jax.experimental.pallas import tpu as pltpu

_TEMPERATURE = 0.1
_INV_T = 1.0 / _TEMPERATURE
_SCALE = 64.0
_DOT_SCALE = _SCALE * _SCALE
_EXP2_C = _INV_T * math.log2(math.e) / _DOT_SCALE
_POS_C = _INV_T / _DOT_SCALE
_EPS = 1e-8
_F8 = jnp.float8_e4m3fn


def _normalize_body(x_ref, xnt8_ref):
    x = x_ref[...]
    nrm = jnp.sqrt(jnp.sum(x * x, axis=1, keepdims=True))
    nrm = jnp.maximum(nrm, _EPS)
    xs = x * (_SCALE / nrm)
    xnt8_ref[...] = xs.astype(_F8).T


def _lane_fold(a, bn):
    acc = a[:, 0:128]
    for k in range(1, bn // 128):
        acc = acc + a[:, k * 128:(k + 1) * 128]
    return acc


def _sublane_fold(a, bm):
    acc = a[0:8, :]
    for k in range(1, bm // 8):
        acc = acc + a[k * 8:(k + 1) * 8, :]
    return acc


def _half2_body(n, cm, bn, sb, rows_ref, cols_ref, dacc_ref, pacc_ref):
    c = pl.program_id(0)
    j = pl.program_id(1)
    for ii in range(cm // sb):
        lo, hi = ii * sb, (ii + 1) * sb
        s = jax.lax.dot_general(
            rows_ref[:, lo:hi], cols_ref[...],
            dimension_numbers=(((0,), (0,)), ((), ())),
            preferred_element_type=jnp.float32,
        )
        e = jnp.exp2(s * _EXP2_C)
        part = _lane_fold(e, bn)
        row_base = c * cm + ii * sb

        @pl.when(j == 0)
        def _init():
            dacc_ref[lo:hi, :] = part
            pacc_ref[lo:hi, :] = jnp.zeros_like(part)

        @pl.when(j != 0)
        def _acc():
            dacc_ref[lo:hi, :] = dacc_ref[lo:hi, :] + part

        @pl.when(j == row_base // bn)
        def _pos():
            row_ids = row_base + jax.lax.broadcasted_iota(jnp.int32, (sb, bn), 0)
            col_ids = (n + j * bn
                       + jax.lax.broadcasted_iota(jnp.int32, (sb, bn), 1))
            masked = jnp.where(col_ids == row_ids + n, s, 0.0)
            pacc_ref[lo:hi, :] = (pacc_ref[lo:hi, :]
                                  + _lane_fold(masked, bn) * _POS_C)


def _pair_ab(t, nblk):
    a = jnp.int32(0)
    for m in range(1, nblk):
        a = a + (t >= m * nblk - (m * (m - 1)) // 2).astype(jnp.int32)
    base = a * nblk - (a * (a - 1)) // 2
    b = t - base + a
    return a, b


def _tri_body(tb, nblk, pp, *refs):
    t = pl.program_id(0)
    ab_refs, (dacc2_ref, cacc_ref) = refs[:2 * pp], refs[2 * pp:]

    @pl.when(t == 0)
    def _init():
        dacc2_ref[...] = jnp.zeros_like(dacc2_ref)
        cacc_ref[...] = jnp.zeros_like(cacc_ref)

    for k in range(pp):
        rows_ref, cols_ref = ab_refs[2 * k], ab_refs[2 * k + 1]
        a, b = _pair_ab(t * pp + k, nblk)
        s = jax.lax.dot_general(
            rows_ref[...], cols_ref[...],
            dimension_numbers=(((0,), (0,)), ((), ())),
            preferred_element_type=jnp.float32,
        )
        e = jnp.exp2(s * _EXP2_C)

        @pl.when(a == b)
        def _diag():
            rr = jax.lax.broadcasted_iota(jnp.int32, (tb, 1), 0)
            cc = jax.lax.broadcasted_iota(jnp.int32, (1, tb), 1)
            part = _lane_fold(jnp.where(rr == cc, 0.0, e), tb)
            dacc2_ref[pl.ds(a, 1), :, :] = (
                dacc2_ref[pl.ds(a, 1), :, :] + part[None, :, :])

        @pl.when(a != b)
        def _off():
            part = _lane_fold(e, tb)
            dacc2_ref[pl.ds(a, 1), :, :] = (
                dacc2_ref[pl.ds(a, 1), :, :] + part[None, :, :])
            cpart = _sublane_fold(e, tb)
            cacc_ref[pl.ds(b, 1), :, :] = (
                cacc_ref[pl.ds(b, 1), :, :] + cpart[None, :, :])


def _loss_body(batch, nblk, tb, dacc_ref, dacc2_ref, pacc_ref, cacc_ref,
               out_ref):
    total = jnp.zeros((1, 1), dtype=jnp.float32)
    csum = jnp.sum(cacc_ref[...], axis=1)
    for bb in range(nblk):
        lo, hi = bb * tb, (bb + 1) * tb
        d = (jnp.sum(dacc_ref[lo:hi, :], axis=1, keepdims=True)
             + jnp.sum(dacc2_ref[bb, :, :], axis=1, keepdims=True))
        d = d + csum[bb:bb + 1, :].T
        p = jnp.sum(pacc_ref[lo:hi, :], axis=1, keepdims=True)
        lt = jnp.log(d) - p
        total = total + jnp.sum(lt, axis=0, keepdims=True)
    out_ref[...] = total * (1.0 / batch)


def kernel(input_val):
    B, D = input_val.shape
    n = B // 2
    RB = min(256, B)
    NC = 2 if n >= 2048 else 1
    CM = n // NC
    SB = min(1024, CM)
    BN = min(2048, n)
    TB = min(1024, n)
    NBLK = n // TB
    NPAIR = NBLK * (NBLK + 1) // 2

    xnt8 = pl.pallas_call(
        _normalize_body,
        grid=(B // RB,),
        in_specs=[pl.BlockSpec((RB, D), lambda i: (i, 0))],
        out_specs=pl.BlockSpec((D, RB), lambda i: (0, i)),
        out_shape=jax.ShapeDtypeStruct((D, B), _F8),
        compiler_params=pltpu.CompilerParams(
            dimension_semantics=("arbitrary",)),
    )(input_val)

    dacc, pacc = pl.pallas_call(
        functools.partial(_half2_body, n, CM, BN, SB),
        grid=(NC, n // BN),
        in_specs=[
            pl.BlockSpec((D, CM), lambda c, j: (0, c)),
            pl.BlockSpec((D, BN), lambda c, j: (0, j + n // BN)),
        ],
        out_specs=[
            pl.BlockSpec((CM, 128), lambda c, j: (c, 0)),
            pl.BlockSpec((CM, 128), lambda c, j: (c, 0)),
        ],
        out_shape=[
            jax.ShapeDtypeStruct((n, 128), jnp.float32),
            jax.ShapeDtypeStruct((n, 128), jnp.float32),
        ],
        compiler_params=pltpu.CompilerParams(
            dimension_semantics=("arbitrary", "arbitrary"),
            vmem_limit_bytes=50 * 1024 * 1024),
    )(xnt8, xnt8)

    PP = 2 if NPAIR % 2 == 0 else 1

    def _a_of(t, k):
        a, _ = _pair_ab(t * PP + k, NBLK)
        return a

    def _b_of(t, k):
        _, b = _pair_ab(t * PP + k, NBLK)
        return b

    tri_in_specs = []
    for k in range(PP):
        tri_in_specs.append(
            pl.BlockSpec((D, TB), functools.partial(
                lambda kk, t: (0, _a_of(t, kk)), k)))
        tri_in_specs.append(
            pl.BlockSpec((D, TB), functools.partial(
                lambda kk, t: (0, _b_of(t, kk)), k)))

    dacc2, cacc = pl.pallas_call(
        functools.partial(_tri_body, TB, NBLK, PP),
        grid=(NPAIR // PP,),
        in_specs=tri_in_specs,
        out_specs=[
            pl.BlockSpec((NBLK, TB, 128), lambda t: (0, 0, 0)),
            pl.BlockSpec((NBLK, 8, TB), lambda t: (0, 0, 0)),
        ],
        out_shape=[
            jax.ShapeDtypeStruct((NBLK, TB, 128), jnp.float32),
            jax.ShapeDtypeStruct((NBLK, 8, TB), jnp.float32),
        ],
        compiler_params=pltpu.CompilerParams(
            dimension_semantics=("arbitrary",),
            vmem_limit_bytes=50 * 1024 * 1024),
    )(*([xnt8] * (2 * PP)))

    loss = pl.pallas_call(
        functools.partial(_loss_body, B, NBLK, TB),
        in_specs=[
            pl.BlockSpec((n, 128), lambda: (0, 0)),
            pl.BlockSpec((NBLK, TB, 128), lambda: (0, 0, 0)),
            pl.BlockSpec((n, 128), lambda: (0, 0)),
            pl.BlockSpec((NBLK, 8, TB), lambda: (0, 0, 0)),
        ],
        out_specs=pl.BlockSpec((1, 1), lambda: (0, 0)),
        out_shape=jax.ShapeDtypeStruct((1, 1), jnp.float32),
    )(dacc, dacc2, pacc, cacc)
    return loss[0, 0]


# speedup vs baseline: 1.0066x; 1.0066x over previous
import functools
import math

import jax
import jax.numpy as jnp
from jax.experimental import pallas as pl
from jax.experimental.pallas import tpu as pltpu

_TEMPERATURE = 0.1
_INV_T = 1.0 / _TEMPERATURE
_SCALE = 64.0
_DOT_SCALE = _SCALE * _SCALE
_EXP2_C = _INV_T * math.log2(math.e) / _DOT_SCALE
_POS_C = _INV_T / _DOT_SCALE
_EPS = 1e-8
_F8 = jnp.float8_e4m3fn


def _normalize_body(x_ref, xnt8_ref):
    x = x_ref[...]
    nrm = jnp.sqrt(jnp.sum(x * x, axis=1, keepdims=True))
    nrm = jnp.maximum(nrm, _EPS)
    xs = x * (_SCALE / nrm)
    xnt8_ref[...] = xs.astype(_F8).T


def _lane_fold(a, bn):
    acc = a[:, 0:128]
    for k in range(1, bn // 128):
        acc = acc + a[:, k * 128:(k + 1) * 128]
    return acc


def _sublane_fold(a, bm):
    acc = a[0:8, :]
    for k in range(1, bm // 8):
        acc = acc + a[k * 8:(k + 1) * 8, :]
    return acc


def _half2_body(n, cm, bn, sb, rows_ref, cols_ref, dacc_ref, pacc_ref):
    c = pl.program_id(0)
    j = pl.program_id(1)
    for ii in range(cm // sb):
        lo, hi = ii * sb, (ii + 1) * sb
        s = jax.lax.dot_general(
            rows_ref[:, lo:hi], cols_ref[...],
            dimension_numbers=(((0,), (0,)), ((), ())),
            preferred_element_type=jnp.float32,
        )
        e = jnp.exp2(s * _EXP2_C)
        part = _lane_fold(e, bn)
        row_base = c * cm + ii * sb

        @pl.when(j == 0)
        def _init():
            dacc_ref[lo:hi, :] = part
            pacc_ref[lo:hi, :] = jnp.zeros_like(part)

        @pl.when(j != 0)
        def _acc():
            dacc_ref[lo:hi, :] = dacc_ref[lo:hi, :] + part

        @pl.when(j == row_base // bn)
        def _pos():
            row_ids = row_base + jax.lax.broadcasted_iota(jnp.int32, (sb, bn), 0)
            col_ids = (n + j * bn
                       + jax.lax.broadcasted_iota(jnp.int32, (sb, bn), 1))
            masked = jnp.where(col_ids == row_ids + n, s, 0.0)
            pacc_ref[lo:hi, :] = (pacc_ref[lo:hi, :]
                                  + _lane_fold(masked, bn) * _POS_C)


def _pair_ab(t, nblk):
    a = jnp.int32(0)
    for m in range(1, nblk):
        a = a + (t >= m * nblk - (m * (m - 1)) // 2).astype(jnp.int32)
    base = a * nblk - (a * (a - 1)) // 2
    b = t - base + a
    return a, b


def _tri_body(tb, nblk, pp, *refs):
    t = pl.program_id(0)
    ab_refs, (dacc2_ref, cacc_ref) = refs[:2 * pp], refs[2 * pp:]

    @pl.when(t == 0)
    def _init():
        dacc2_ref[...] = jnp.zeros_like(dacc2_ref)
        cacc_ref[...] = jnp.zeros_like(cacc_ref)

    for k in range(pp):
        rows_ref, cols_ref = ab_refs[2 * k], ab_refs[2 * k + 1]
        a, b = _pair_ab(t * pp + k, nblk)
        s = jax.lax.dot_general(
            rows_ref[...], cols_ref[...],
            dimension_numbers=(((0,), (0,)), ((), ())),
            preferred_element_type=jnp.float32,
        )
        e = jnp.exp2(s * _EXP2_C)

        @pl.when(a == b)
        def _diag():
            rr = jax.lax.broadcasted_iota(jnp.int32, (tb, 1), 0)
            cc = jax.lax.broadcasted_iota(jnp.int32, (1, tb), 1)
            part = _lane_fold(jnp.where(rr == cc, 0.0, e), tb)
            dacc2_ref[pl.ds(a, 1), :, :] = (
                dacc2_ref[pl.ds(a, 1), :, :] + part[None, :, :])

        @pl.when(a != b)
        def _off():
            part = _lane_fold(e, tb)
            dacc2_ref[pl.ds(a, 1), :, :] = (
                dacc2_ref[pl.ds(a, 1), :, :] + part[None, :, :])
            cpart = _sublane_fold(e, tb)
            cacc_ref[pl.ds(b, 1), :, :] = (
                cacc_ref[pl.ds(b, 1), :, :] + cpart[None, :, :])


def _loss_body(batch, nblk, tb, dacc_ref, dacc2_ref, pacc_ref, cacc_ref,
               out_ref):
    total = jnp.zeros((1, 1), dtype=jnp.float32)
    csum = jnp.sum(cacc_ref[...], axis=1)
    for bb in range(nblk):
        lo, hi = bb * tb, (bb + 1) * tb
        d = (jnp.sum(dacc_ref[lo:hi, :], axis=1, keepdims=True)
             + jnp.sum(dacc2_ref[bb, :, :], axis=1, keepdims=True))
        d = d + csum[bb:bb + 1, :].T
        p = jnp.sum(pacc_ref[lo:hi, :], axis=1, keepdims=True)
        lt = jnp.log(d) - p
        total = total + jnp.sum(lt, axis=0, keepdims=True)
    out_ref[...] = total * (1.0 / batch)


def kernel(input_val):
    B, D = input_val.shape
    n = B // 2
    RB = min(256, B)
    NC = 2 if n >= 2048 else 1
    CM = n // NC
    SB = min(1024, CM)
    BN = min(2048, n)
    TB = min(1024, n)
    NBLK = n // TB
    NPAIR = NBLK * (NBLK + 1) // 2

    xnt8 = pl.pallas_call(
        _normalize_body,
        grid=(B // RB,),
        in_specs=[pl.BlockSpec((RB, D), lambda i: (i, 0))],
        out_specs=pl.BlockSpec((D, RB), lambda i: (0, i)),
        out_shape=jax.ShapeDtypeStruct((D, B), _F8),
        compiler_params=pltpu.CompilerParams(
            dimension_semantics=("arbitrary",)),
    )(input_val)

    dacc, pacc = pl.pallas_call(
        functools.partial(_half2_body, n, CM, BN, SB),
        grid=(NC, n // BN),
        in_specs=[
            pl.BlockSpec((D, CM), lambda c, j: (0, c)),
            pl.BlockSpec((D, BN), lambda c, j: (0, j + n // BN)),
        ],
        out_specs=[
            pl.BlockSpec((CM, 128), lambda c, j: (c, 0)),
            pl.BlockSpec((CM, 128), lambda c, j: (c, 0)),
        ],
        out_shape=[
            jax.ShapeDtypeStruct((n, 128), jnp.float32),
            jax.ShapeDtypeStruct((n, 128), jnp.float32),
        ],
        compiler_params=pltpu.CompilerParams(
            dimension_semantics=("arbitrary", "arbitrary"),
            vmem_limit_bytes=50 * 1024 * 1024),
    )(xnt8, xnt8)

    PP = 1

    def _a_of(t, k):
        a, _ = _pair_ab(t * PP + k, NBLK)
        return a

    def _b_of(t, k):
        _, b = _pair_ab(t * PP + k, NBLK)
        return b

    tri_in_specs = []
    for k in range(PP):
        tri_in_specs.append(
            pl.BlockSpec((D, TB), functools.partial(
                lambda kk, t: (0, _a_of(t, kk)), k)))
        tri_in_specs.append(
            pl.BlockSpec((D, TB), functools.partial(
                lambda kk, t: (0, _b_of(t, kk)), k)))

    dacc2, cacc = pl.pallas_call(
        functools.partial(_tri_body, TB, NBLK, PP),
        grid=(NPAIR // PP,),
        in_specs=tri_in_specs,
        out_specs=[
            pl.BlockSpec((NBLK, TB, 128), lambda t: (0, 0, 0)),
            pl.BlockSpec((NBLK, 8, TB), lambda t: (0, 0, 0)),
        ],
        out_shape=[
            jax.ShapeDtypeStruct((NBLK, TB, 128), jnp.float32),
            jax.ShapeDtypeStruct((NBLK, 8, TB), jnp.float32),
        ],
        compiler_params=pltpu.CompilerParams(
            dimension_semantics=("arbitrary",),
            vmem_limit_bytes=50 * 1024 * 1024),
    )(*([xnt8] * (2 * PP)))

    loss = pl.pallas_call(
        functools.partial(_loss_body, B, NBLK, TB),
        in_specs=[
            pl.BlockSpec((n, 128), lambda: (0, 0)),
            pl.BlockSpec((NBLK, TB, 128), lambda: (0, 0, 0)),
            pl.BlockSpec((n, 128), lambda: (0, 0)),
            pl.BlockSpec((NBLK, 8, TB), lambda: (0, 0, 0)),
        ],
        out_specs=pl.BlockSpec((1, 1), lambda: (0, 0)),
        out_shape=jax.ShapeDtypeStruct((1, 1), jnp.float32),
    )(dacc, dacc2, pacc, cacc)
    return loss[0, 0]
